# baseline (speedup 1.0000x reference)
import functools

import jax
import jax.numpy as jnp
from jax import lax
from jax.experimental import pallas as pl
from jax.experimental.pallas import tpu as pltpu
from jax.experimental.pallas import tpu_sc as plsc

_H = 128
_D = 64
_V = 1024


def _pos_encoding(seq_len, d_model):
    pos = jnp.arange(seq_len, dtype=jnp.float32)[:, None]
    dim = jnp.arange(0, d_model, 2, dtype=jnp.float32)
    angle = pos / jnp.power(10000.0, dim / float(d_model))
    res = jnp.zeros((seq_len, d_model), dtype=jnp.float32)
    res = res.at[:, 0::2].set(jnp.sin(angle))
    res = res.at[:, 1::2].set(jnp.cos(angle))
    return res


def kernel(target, Wx, Wy):
    B, T, N, _ = target.shape
    K = B * N * T * 2
    P = 2 * T

    ws = jnp.concatenate([Wx, Wy], axis=0)
    pe2 = _pos_encoding(T, _H).reshape(P, _D)
    flat_idx = (
        jnp.transpose(target, (0, 2, 1, 3))
        + jnp.array([0, _V], dtype=target.dtype)
    ).reshape(K).astype(jnp.int32)

    info = plsc.get_sparse_core_info()
    nw = info.num_cores * info.num_subcores
    per_w = K // nw
    ch = 4 * P
    n_ch = per_w // ch

    mesh = plsc.VectorSubcoreMesh(core_axis_name="c", subcore_axis_name="s")

    @functools.partial(
        pl.kernel,
        out_type=jax.ShapeDtypeStruct((K, _D), jnp.float32),
        mesh=mesh,
        compiler_params=pltpu.CompilerParams(
            use_tc_tiling_on_sc=False, needs_layout_passes=False),
        scratch_types=(
            pltpu.VMEM((ch,), jnp.int32),
            pltpu.VMEM((ch, _D), jnp.float32),
            pltpu.VMEM((P, _D), jnp.float32),
            pltpu.SemaphoreType.DMA,
        ),
    )
    def emb_kernel(ws_hbm, idx_hbm, pe_hbm, out_hbm, idx_v, gbuf, pe_v, sem):
        wid = lax.axis_index("s") * info.num_cores + lax.axis_index("c")
        base = wid * per_w
        pltpu.sync_copy(pe_hbm, pe_v)

        @pl.loop(0, n_ch)
        def _chunk(ci):
            start = base + ci * ch
            pltpu.sync_copy(idx_hbm.at[pl.ds(start, ch)], idx_v)
            pltpu.async_copy(ws_hbm.at[idx_v], gbuf, sem).wait()

            for rep in range(ch // P):
                @pl.loop(0, P)
                def _row(r):
                    for q in range(_D // 16):
                        sl = pl.ds(q * 16, 16)
                        gbuf[rep * P + r, sl] = gbuf[rep * P + r, sl] + pe_v[r, sl]

            pltpu.sync_copy(gbuf, out_hbm.at[pl.ds(start, ch)])

    out = emb_kernel(ws, flat_idx, pe2)
    return out.reshape(B * N, T, _H)


# speedup vs baseline: 1.1050x; 1.1050x over previous
import functools

import jax
import jax.numpy as jnp
from jax import lax
from jax.experimental import pallas as pl
from jax.experimental.pallas import tpu as pltpu
from jax.experimental.pallas import tpu_sc as plsc

_H = 128
_D = 64
_V = 1024


def _pos_encoding(seq_len, d_model):
    pos = jnp.arange(seq_len, dtype=jnp.float32)[:, None]
    dim = jnp.arange(0, d_model, 2, dtype=jnp.float32)
    angle = pos / jnp.power(10000.0, dim / float(d_model))
    res = jnp.zeros((seq_len, d_model), dtype=jnp.float32)
    res = res.at[:, 0::2].set(jnp.sin(angle))
    res = res.at[:, 1::2].set(jnp.cos(angle))
    return res


def kernel(target, Wx, Wy):
    B, T, N, _ = target.shape
    K = B * N * T * 2
    P = 2 * T

    ws = jnp.concatenate([Wx, Wy], axis=0)
    pe2 = _pos_encoding(T, _H).reshape(P, _D)
    flat_idx = (
        jnp.transpose(target, (0, 2, 1, 3))
        + jnp.array([0, _V], dtype=target.dtype)
    ).reshape(K).astype(jnp.int32)

    info = plsc.get_sparse_core_info()
    nw = info.num_cores * info.num_subcores
    per_w = K // nw
    ch = 2 * P
    n_ch = per_w // ch
    nbuf = 2

    mesh = plsc.VectorSubcoreMesh(core_axis_name="c", subcore_axis_name="s")

    @functools.partial(
        pl.kernel,
        out_type=jax.ShapeDtypeStruct((K, _D), jnp.float32),
        mesh=mesh,
        compiler_params=pltpu.CompilerParams(
            use_tc_tiling_on_sc=False, needs_layout_passes=False),
        scratch_types=(
            [pltpu.VMEM((ch,), jnp.int32) for _ in range(nbuf)]
            + [pltpu.VMEM((ch, _D), jnp.float32) for _ in range(nbuf)]
            + [pltpu.VMEM((P, _D), jnp.float32)]
            + [pltpu.SemaphoreType.DMA for _ in range(nbuf)]
        ),
    )
    def emb_kernel(ws_hbm, idx_hbm, pe_hbm, out_hbm, *refs):
        idx_v = refs[:nbuf]
        gbuf = refs[nbuf:2 * nbuf]
        pe_v = refs[2 * nbuf]
        sem = refs[2 * nbuf + 1:]

        wid = lax.axis_index("s") * info.num_cores + lax.axis_index("c")
        base = wid * per_w
        pltpu.sync_copy(pe_hbm, pe_v)

        def start_gather(ci, b):
            pltpu.sync_copy(idx_hbm.at[pl.ds(base + ci * ch, ch)], idx_v[b])
            pltpu.async_copy(ws_hbm.at[idx_v[b]], gbuf[b], sem[b])

        for b in range(nbuf):
            start_gather(b, b)

        @pl.loop(0, n_ch // nbuf)
        def _ring(g):
            for b in range(nbuf):
                ci = g * nbuf + b
                pltpu.make_async_copy(ws_hbm.at[idx_v[b]], gbuf[b], sem[b]).wait()

                for rep in range(ch // P):
                    @pl.loop(0, P)
                    def _row(r):
                        for q in range(_D // 16):
                            sl = pl.ds(q * 16, 16)
                            gbuf[b][rep * P + r, sl] = (
                                gbuf[b][rep * P + r, sl] + pe_v[r, sl])

                pltpu.sync_copy(gbuf[b], out_hbm.at[pl.ds(base + ci * ch, ch)])

                @pl.when(ci + nbuf < n_ch)
                def _next():
                    start_gather(ci + nbuf, b)

    out = emb_kernel(ws, flat_idx, pe2)
    return out.reshape(B * N, T, _H)


# speedup vs baseline: 1.1221x; 1.0155x over previous
import functools

import jax
import jax.numpy as jnp
from jax import lax
from jax.experimental import pallas as pl
from jax.experimental.pallas import tpu as pltpu
from jax.experimental.pallas import tpu_sc as plsc

_H = 128
_D = 64
_V = 1024


def _pos_encoding(seq_len, d_model):
    pos = jnp.arange(seq_len, dtype=jnp.float32)[:, None]
    dim = jnp.arange(0, d_model, 2, dtype=jnp.float32)
    angle = pos / jnp.power(10000.0, dim / float(d_model))
    res = jnp.zeros((seq_len, d_model), dtype=jnp.float32)
    res = res.at[:, 0::2].set(jnp.sin(angle))
    res = res.at[:, 1::2].set(jnp.cos(angle))
    return res


def kernel(target, Wx, Wy):
    B, T, N, _ = target.shape
    K = B * N * T * 2
    P = 2 * T

    ws = jnp.concatenate([Wx, Wy], axis=0)
    pe2 = _pos_encoding(T, _H).reshape(P, _D)
    flat_idx = (
        jnp.transpose(target, (0, 2, 1, 3))
        + jnp.array([0, _V], dtype=target.dtype)
    ).reshape(K).astype(jnp.int32)

    info = plsc.get_sparse_core_info()
    nw = info.num_cores * info.num_subcores
    per_w = K // nw
    ch = 2 * P
    n_ch = per_w // ch
    nbuf = 2

    mesh = plsc.VectorSubcoreMesh(core_axis_name="c", subcore_axis_name="s")

    @functools.partial(
        pl.kernel,
        out_type=jax.ShapeDtypeStruct((K, _D), jnp.float32),
        mesh=mesh,
        compiler_params=pltpu.CompilerParams(
            use_tc_tiling_on_sc=False, needs_layout_passes=False),
        scratch_types=(
            [pltpu.VMEM((ch,), jnp.int32) for _ in range(nbuf)]
            + [pltpu.VMEM((ch, _D), jnp.float32) for _ in range(nbuf)]
            + [pltpu.VMEM((ch, _D), jnp.float32) for _ in range(nbuf)]
            + [pltpu.VMEM((P, _D), jnp.float32)]
            + [pltpu.SemaphoreType.DMA for _ in range(2 * nbuf)]
        ),
    )
    def emb_kernel(ws_hbm, idx_hbm, pe_hbm, out_hbm, *refs):
        idx_v = refs[:nbuf]
        gbuf = refs[nbuf:2 * nbuf]
        obuf = refs[2 * nbuf:3 * nbuf]
        pe_v = refs[3 * nbuf]
        gsem = refs[3 * nbuf + 1:3 * nbuf + 1 + nbuf]
        wsem = refs[3 * nbuf + 1 + nbuf:]

        wid = lax.axis_index("s") * info.num_cores + lax.axis_index("c")
        base = wid * per_w
        pltpu.sync_copy(pe_hbm, pe_v)

        def start_gather(ci, b):
            pltpu.sync_copy(idx_hbm.at[pl.ds(base + ci * ch, ch)], idx_v[b])
            pltpu.async_copy(ws_hbm.at[idx_v[b]], gbuf[b], gsem[b])

        def wait_write(ci, b):
            pltpu.make_async_copy(
                obuf[b], out_hbm.at[pl.ds(base + ci * ch, ch)], wsem[b]).wait()

        for b in range(nbuf):
            start_gather(b, b)

        @pl.loop(0, n_ch // nbuf)
        def _ring(g):
            for b in range(nbuf):
                ci = g * nbuf + b
                pltpu.make_async_copy(ws_hbm.at[idx_v[b]], gbuf[b], gsem[b]).wait()

                @pl.when(ci >= nbuf)
                def _drain():
                    wait_write(ci - nbuf, b)

                for rep in range(ch // P):
                    @pl.loop(0, P)
                    def _row(r):
                        for q in range(_D // 16):
                            sl = pl.ds(q * 16, 16)
                            obuf[b][rep * P + r, sl] = (
                                gbuf[b][rep * P + r, sl] + pe_v[r, sl])

                pltpu.async_copy(
                    obuf[b], out_hbm.at[pl.ds(base + ci * ch, ch)], wsem[b])

                @pl.when(ci + nbuf < n_ch)
                def _next():
                    start_gather(ci + nbuf, b)

        for b in range(nbuf):
            wait_write(n_ch - nbuf + b, b)

    out = emb_kernel(ws, flat_idx, pe2)
    return out.reshape(B * N, T, _H)


# speedup vs baseline: 1.1263x; 1.0038x over previous
import functools

import jax
import jax.numpy as jnp
from jax import lax
from jax.experimental import pallas as pl
from jax.experimental.pallas import tpu as pltpu
from jax.experimental.pallas import tpu_sc as plsc

_H = 128
_D = 64
_V = 1024


def _pos_encoding(seq_len, d_model):
    pos = jnp.arange(seq_len, dtype=jnp.float32)[:, None]
    dim = jnp.arange(0, d_model, 2, dtype=jnp.float32)
    angle = pos / jnp.power(10000.0, dim / float(d_model))
    res = jnp.zeros((seq_len, d_model), dtype=jnp.float32)
    res = res.at[:, 0::2].set(jnp.sin(angle))
    res = res.at[:, 1::2].set(jnp.cos(angle))
    return res


def kernel(target, Wx, Wy):
    B, T, N, _ = target.shape
    K = B * N * T * 2
    P = 2 * T

    ws = jnp.concatenate([Wx, Wy], axis=0)
    pe2 = _pos_encoding(T, _H).reshape(P, _D)
    flat_idx = (
        jnp.transpose(target, (0, 2, 1, 3))
        + jnp.array([0, _V], dtype=target.dtype)
    ).reshape(K).astype(jnp.int32)

    info = plsc.get_sparse_core_info()
    nw = info.num_cores * info.num_subcores
    per_w = K // nw
    ch = 2 * P
    n_ch = per_w // ch
    nbuf = 2

    mesh = plsc.VectorSubcoreMesh(core_axis_name="c", subcore_axis_name="s")

    @functools.partial(
        pl.kernel,
        out_type=jax.ShapeDtypeStruct((K, _D), jnp.float32),
        mesh=mesh,
        compiler_params=pltpu.CompilerParams(
            use_tc_tiling_on_sc=False, needs_layout_passes=False),
        scratch_types=(
            [pltpu.VMEM((per_w,), jnp.int32)]
            + [pltpu.VMEM((ch, _D), jnp.float32) for _ in range(nbuf)]
            + [pltpu.VMEM((ch, _D), jnp.float32) for _ in range(nbuf)]
            + [pltpu.VMEM((P, _D), jnp.float32)]
            + [pltpu.SemaphoreType.DMA for _ in range(2 * nbuf)]
        ),
    )
    def emb_kernel(ws_hbm, idx_hbm, pe_hbm, out_hbm, *refs):
        idx_v = refs[0]
        gbuf = refs[1:1 + nbuf]
        obuf = refs[1 + nbuf:1 + 2 * nbuf]
        pe_v = refs[1 + 2 * nbuf]
        gsem = refs[2 + 2 * nbuf:2 + 3 * nbuf]
        wsem = refs[2 + 3 * nbuf:]

        wid = lax.axis_index("s") * info.num_cores + lax.axis_index("c")
        base = wid * per_w
        pltpu.sync_copy(idx_hbm.at[pl.ds(base, per_w)], idx_v)
        pltpu.sync_copy(pe_hbm, pe_v)

        def start_gather(ci, b):
            pltpu.async_copy(
                ws_hbm.at[idx_v.at[pl.ds(ci * ch, ch)]], gbuf[b], gsem[b])

        def wait_write(ci, b):
            pltpu.make_async_copy(
                obuf[b], out_hbm.at[pl.ds(base + ci * ch, ch)], wsem[b]).wait()

        for b in range(nbuf):
            start_gather(b, b)

        @pl.loop(0, n_ch // nbuf)
        def _ring(g):
            for b in range(nbuf):
                ci = g * nbuf + b
                pltpu.make_async_copy(
                    ws_hbm.at[idx_v.at[pl.ds(ci * ch, ch)]], gbuf[b],
                    gsem[b]).wait()

                @pl.when(ci >= nbuf)
                def _drain():
                    wait_write(ci - nbuf, b)

                for rep in range(ch // P):
                    @pl.loop(0, P)
                    def _row(r):
                        for q in range(_D // 16):
                            sl = pl.ds(q * 16, 16)
                            obuf[b][rep * P + r, sl] = (
                                gbuf[b][rep * P + r, sl] + pe_v[r, sl])

                pltpu.async_copy(
                    obuf[b], out_hbm.at[pl.ds(base + ci * ch, ch)], wsem[b])

                @pl.when(ci + nbuf < n_ch)
                def _next():
                    start_gather(ci + nbuf, b)

        for b in range(nbuf):
            wait_write(n_ch - nbuf + b, b)

    out = emb_kernel(ws, flat_idx, pe2)
    return out.reshape(B * N, T, _H)
